# baseline (speedup 1.0000x reference)
import jax
import jax.numpy as jnp
from jax import lax
from jax.experimental import pallas as pl
from jax.experimental.pallas import tpu as pltpu
from jax.experimental.pallas import tpu_sc as plsc

N = 160000
NEI = 16
H = 128

NC = 2
NS = 16
NW = NC * NS

NODES_PER_W = N // NW
CN = 8
RC = CN * NEI
NCH = NODES_PER_W // CN


def _gather_sum_body(msg_hbm, mg_hbm, out_hbm,
                     idx_a, idx_b, idx_c, idx_d,
                     rows_a, rows_b, rows_c, rows_d, out_a, out_b,
                     si_a, si_b, si_c, si_d,
                     sg_a, sg_b, sg_c, sg_d, so_a, so_b):
  w = lax.axis_index("s") * NC + lax.axis_index("c")
  base = w * NODES_PER_W
  idx = (idx_a, idx_b, idx_c, idx_d)
  rows = (rows_a, rows_b, rows_c, rows_d)
  out = (out_a, out_b)
  si = (si_a, si_b, si_c, si_d)
  sg = (sg_a, sg_b, sg_c, sg_d)
  so = (so_a, so_b)

  def _idx_args(c, t):
    return (mg_hbm.at[pl.ds((base + c * CN) * NEI, RC)], idx[t], si[t])

  def _gather_args(t):
    return (msg_hbm.at[idx[t]], rows[t], sg[t])

  def _out_args(c, t):
    return (out[t], out_hbm.at[pl.ds(base + c * CN, CN)], so[t])

  def fire_idx(c, t):
    return pltpu.async_copy(*_idx_args(c, t))

  def wait_idx(c, t):
    pltpu.make_async_copy(*_idx_args(c, t)).wait()

  def fire_gather(t):
    return pltpu.async_copy(*_gather_args(t))

  def wait_gather(t):
    pltpu.make_async_copy(*_gather_args(t)).wait()

  def fire_out(c, t):
    return pltpu.async_copy(*_out_args(c, t))

  def wait_out(c, t):
    pltpu.make_async_copy(*_out_args(c, t)).wait()

  def reduce_chunk(t, ot):
    def node_body(n, _):
      r0 = n * NEI
      for v in range(H // 16):
        acc = rows[t][r0, pl.ds(v * 16, 16)]
        for j in range(1, NEI):
          acc = acc + rows[t][r0 + j, pl.ds(v * 16, 16)]
        out[ot][n, pl.ds(v * 16, 16)] = acc
      return 0

    lax.fori_loop(0, CN, node_body, 0, unroll=False)

  def body(c, t):
    ot = t % 2
    wait_gather(t)

    @pl.when(c <= NCH - 5)
    def _():
      fire_idx(c + 4, t)

    @pl.when(c <= NCH - 3)
    def _():
      wait_idx(c + 2, (t + 2) % 4)
      fire_gather((t + 2) % 4)

    @pl.when(c >= 2)
    def _():
      wait_out(c - 2, ot)

    reduce_chunk(t, ot)
    fire_out(c, ot)

  fire_idx(0, 0).wait()
  fire_idx(1, 1).wait()
  fire_idx(2, 2)
  fire_idx(3, 3)
  fire_gather(0)
  fire_gather(1)

  def quad_body(cc, _):
    body(4 * cc, 0)
    for t in range(1, 4):
      @pl.when(4 * cc + t <= NCH - 1)
      def _(t=t):
        body(4 * cc + t, t)

    return 0

  lax.fori_loop(0, (NCH + 3) // 4, quad_body, 0, unroll=False)

  wait_out(NCH - 2, (NCH - 2) % 2)
  wait_out(NCH - 1, (NCH - 1) % 2)


def _gather_sum(messages, mg_flat):
  mesh = plsc.VectorSubcoreMesh(core_axis_name="c", subcore_axis_name="s")
  return pl.kernel(
      _gather_sum_body,
      out_type=jax.ShapeDtypeStruct((N, H), jnp.float32),
      mesh=mesh,
      compiler_params=pltpu.CompilerParams(needs_layout_passes=False),
      scratch_types=(
          [pltpu.VMEM((RC,), jnp.int32)] * 4
          + [pltpu.VMEM((RC, H), jnp.float32)] * 4
          + [pltpu.VMEM((CN, H), jnp.float32)] * 2
          + [pltpu.SemaphoreType.DMA] * 10
      ),
  )(messages, mg_flat)


BR = 1280
NBLK = N // BR


def _mlp0_body(fm, wg1a, bg1, wg2, bg2, tmp):
  h1 = jnp.maximum(
      jnp.dot(fm[...], wg1a[...], preferred_element_type=jnp.float32)
      + bg1[...], 0.0)
  t = jnp.dot(h1, wg2[...], preferred_element_type=jnp.float32) + bg2[...]
  rid = pl.program_id(0) * BR + lax.broadcasted_iota(jnp.int32, (BR, H), 0)
  tmp[...] = jnp.where(rid == 0, 0.0, t)


def _mlp_body(fm, nei, wg1a, wg1b, bg1, wg2, bg2, tmp):
  h1 = jnp.maximum(
      jnp.dot(fm[...], wg1a[...], preferred_element_type=jnp.float32)
      + jnp.dot(nei[...], wg1b[...], preferred_element_type=jnp.float32)
      + bg1[...], 0.0)
  t = jnp.dot(h1, wg2[...], preferred_element_type=jnp.float32) + bg2[...]
  rid = pl.program_id(0) * BR + lax.broadcasted_iota(jnp.int32, (BR, H), 0)
  tmp[...] = jnp.where(rid == 0, 0.0, t)


def _out_body(t0, t1, t2, t3, wo1, bo1, wo2, bo2, out):
  z = jnp.dot(t0[...], wo1[0], preferred_element_type=jnp.float32)
  z += jnp.dot(t1[...], wo1[1], preferred_element_type=jnp.float32)
  z += jnp.dot(t2[...], wo1[2], preferred_element_type=jnp.float32)
  z += jnp.dot(t3[...], wo1[3], preferred_element_type=jnp.float32)
  h2 = jnp.maximum(z + bo1[...], 0.0)
  o = jnp.dot(h2, wo2[...], preferred_element_type=jnp.float32) + bo2[...]
  rid = pl.program_id(0) * BR + lax.broadcasted_iota(jnp.int32, (BR, H), 0)
  out[...] = jnp.where(rid == 0, 0.0, o)


def _row_spec(width, dtype=None):
  return pl.BlockSpec((BR, width), lambda i: (i, 0))


def _full_spec(shape):
  return pl.BlockSpec(shape, lambda i: tuple(0 for _ in shape))


def kernel(fmess, mess_graph, Wg1, bg1, Wg2, bg2, Wo1, bo1, Wo2, bo2):
  wg1a = Wg1[:H]
  wg1b = Wg1[H:]
  wo1 = Wo1.reshape(4, H, 2 * H)
  mg_flat = mess_graph.reshape(-1)

  f32 = jax.ShapeDtypeStruct((N, H), jnp.float32)

  mlp0 = pl.pallas_call(
      _mlp0_body,
      grid=(NBLK,),
      in_specs=[_row_spec(H), _full_spec((H, H)), _full_spec((H,)),
                _full_spec((H, H)), _full_spec((H,))],
      out_specs=_row_spec(H),
      out_shape=f32,
  )
  tmp0 = mlp0(fmess, wg1a, bg1, Wg2, bg2)

  mlp = pl.pallas_call(
      _mlp_body,
      grid=(NBLK,),
      in_specs=[_row_spec(H), _row_spec(H), _full_spec((H, H)),
                _full_spec((H, H)), _full_spec((H,)),
                _full_spec((H, H)), _full_spec((H,))],
      out_specs=_row_spec(H),
      out_shape=f32,
  )

  tmps = [tmp0]
  for d in range(1, 4):
    nei = _gather_sum(tmps[-1], mg_flat)
    tmps.append(mlp(fmess, nei, wg1a, wg1b, bg1, Wg2, bg2))

  out_mlp = pl.pallas_call(
      _out_body,
      grid=(NBLK,),
      in_specs=[_row_spec(H), _row_spec(H), _row_spec(H), _row_spec(H),
                _full_spec((4, H, 2 * H)), _full_spec((2 * H,)),
                _full_spec((2 * H, H)), _full_spec((H,))],
      out_specs=_row_spec(H),
      out_shape=f32,
  )
  return out_mlp(tmps[0], tmps[1], tmps[2], tmps[3], wo1, bo1, Wo2, bo2)


# speedup vs baseline: 1.1118x; 1.1118x over previous
import jax
import jax.numpy as jnp
from jax import lax
from jax.experimental import pallas as pl
from jax.experimental.pallas import tpu as pltpu
from jax.experimental.pallas import tpu_sc as plsc

N = 160000
NEI = 16
H = 128

NC = 2
NS = 16
NW = NC * NS

NODES_PER_W = N // NW
CN = 8
RC = CN * NEI
NCH = NODES_PER_W // CN


def _gather_sum_body(msg_hbm, mg_hbm, out_hbm,
                     idx_a, idx_b, rows_a, rows_b, out_a, out_b,
                     si_a, si_b, sg_a, sg_b, so_a, so_b):
  w = lax.axis_index("s") * NC + lax.axis_index("c")
  base = w * NODES_PER_W
  idx = (idx_a, idx_b)
  rows = (rows_a, rows_b)
  out = (out_a, out_b)
  si = (si_a, si_b)
  sg = (sg_a, sg_b)
  so = (so_a, so_b)

  def _idx_args(c, t):
    return (mg_hbm.at[pl.ds((base + c * CN) * NEI, RC)], idx[t], si[t])

  def _gather_args(t):
    return (msg_hbm.at[idx[t]], rows[t], sg[t])

  def _out_args(c, t):
    return (out[t], out_hbm.at[pl.ds(base + c * CN, CN)], so[t])

  def fire_idx(c, t):
    return pltpu.async_copy(*_idx_args(c, t))

  def wait_idx(c, t):
    pltpu.make_async_copy(*_idx_args(c, t)).wait()

  def fire_gather(t):
    return pltpu.async_copy(*_gather_args(t))

  def wait_gather(t):
    pltpu.make_async_copy(*_gather_args(t)).wait()

  def fire_out(c, t):
    return pltpu.async_copy(*_out_args(c, t))

  def wait_out(c, t):
    pltpu.make_async_copy(*_out_args(c, t)).wait()

  def reduce_chunk(t):
    def node_body(n, _):
      r0 = n * NEI
      for v in range(H // 16):
        acc = rows[t][r0, pl.ds(v * 16, 16)]
        for j in range(1, NEI):
          acc = acc + rows[t][r0 + j, pl.ds(v * 16, 16)]
        out[t][n, pl.ds(v * 16, 16)] = acc
      return 0

    lax.fori_loop(0, CN, node_body, 0, unroll=False)

  def body(c, t):
    o = 1 - t
    wait_gather(t)

    @pl.when(c <= NCH - 3)
    def _():
      fire_idx(c + 2, t)

    @pl.when(c <= NCH - 2)
    def _():
      wait_idx(c + 1, o)
      fire_gather(o)

    @pl.when(c >= 2)
    def _():
      wait_out(c - 2, t)

    reduce_chunk(t)
    fire_out(c, t)

  fire_idx(0, 0).wait()
  fire_idx(1, 1)
  fire_gather(0)

  def pair_body(cc, _):
    body(2 * cc, 0)

    @pl.when(2 * cc + 1 <= NCH - 1)
    def _():
      body(2 * cc + 1, 1)

    return 0

  lax.fori_loop(0, (NCH + 1) // 2, pair_body, 0, unroll=False)

  wait_out(NCH - 2, (NCH - 2) % 2)
  wait_out(NCH - 1, (NCH - 1) % 2)


def _gather_sum(messages, mg_flat):
  mesh = plsc.VectorSubcoreMesh(core_axis_name="c", subcore_axis_name="s")
  return pl.kernel(
      _gather_sum_body,
      out_type=jax.ShapeDtypeStruct((N, H), jnp.float32),
      mesh=mesh,
      compiler_params=pltpu.CompilerParams(needs_layout_passes=False),
      scratch_types=(
          [pltpu.VMEM((RC,), jnp.int32)] * 2
          + [pltpu.VMEM((RC, H), jnp.float32)] * 2
          + [pltpu.VMEM((CN, H), jnp.float32)] * 2
          + [pltpu.SemaphoreType.DMA] * 6
      ),
  )(messages, mg_flat)


BR = 1280
NBLK = N // BR


def _bdot(a, b):
  return jnp.dot(a.astype(jnp.bfloat16), b.astype(jnp.bfloat16),
                 preferred_element_type=jnp.float32)


def _mlp0_body(fm, wg1a, bg1, wg2, bg2, tmp):
  h1 = jnp.maximum(_bdot(fm[...], wg1a[...]) + bg1[...], 0.0)
  t = _bdot(h1, wg2[...]) + bg2[...]
  rid = pl.program_id(0) * BR + lax.broadcasted_iota(jnp.int32, (BR, H), 0)
  tmp[...] = jnp.where(rid == 0, 0.0, t)


def _mlp_body(fm, nei, wg1a, wg1b, bg1, wg2, bg2, tmp):
  h1 = jnp.maximum(
      _bdot(fm[...], wg1a[...]) + _bdot(nei[...], wg1b[...]) + bg1[...], 0.0)
  t = _bdot(h1, wg2[...]) + bg2[...]
  rid = pl.program_id(0) * BR + lax.broadcasted_iota(jnp.int32, (BR, H), 0)
  tmp[...] = jnp.where(rid == 0, 0.0, t)


def _out_body(t0, t1, t2, t3, wo1, bo1, wo2, bo2, out):
  z = _bdot(t0[...], wo1[0])
  z += _bdot(t1[...], wo1[1])
  z += _bdot(t2[...], wo1[2])
  z += _bdot(t3[...], wo1[3])
  h2 = jnp.maximum(z + bo1[...], 0.0)
  o = _bdot(h2, wo2[...]) + bo2[...]
  rid = pl.program_id(0) * BR + lax.broadcasted_iota(jnp.int32, (BR, H), 0)
  out[...] = jnp.where(rid == 0, 0.0, o)


def _row_spec(width, dtype=None):
  return pl.BlockSpec((BR, width), lambda i: (i, 0))


def _full_spec(shape):
  return pl.BlockSpec(shape, lambda i: tuple(0 for _ in shape))


def kernel(fmess, mess_graph, Wg1, bg1, Wg2, bg2, Wo1, bo1, Wo2, bo2):
  wg1a = Wg1[:H]
  wg1b = Wg1[H:]
  wo1 = Wo1.reshape(4, H, 2 * H)
  mg_flat = mess_graph.reshape(-1)

  f32 = jax.ShapeDtypeStruct((N, H), jnp.float32)

  mlp0 = pl.pallas_call(
      _mlp0_body,
      grid=(NBLK,),
      in_specs=[_row_spec(H), _full_spec((H, H)), _full_spec((H,)),
                _full_spec((H, H)), _full_spec((H,))],
      out_specs=_row_spec(H),
      out_shape=f32,
  )
  tmp0 = mlp0(fmess, wg1a, bg1, Wg2, bg2)

  mlp = pl.pallas_call(
      _mlp_body,
      grid=(NBLK,),
      in_specs=[_row_spec(H), _row_spec(H), _full_spec((H, H)),
                _full_spec((H, H)), _full_spec((H,)),
                _full_spec((H, H)), _full_spec((H,))],
      out_specs=_row_spec(H),
      out_shape=f32,
  )

  tmps = [tmp0]
  for d in range(1, 4):
    nei = _gather_sum(tmps[-1], mg_flat)
    tmps.append(mlp(fmess, nei, wg1a, wg1b, bg1, Wg2, bg2))

  out_mlp = pl.pallas_call(
      _out_body,
      grid=(NBLK,),
      in_specs=[_row_spec(H), _row_spec(H), _row_spec(H), _row_spec(H),
                _full_spec((4, H, 2 * H)), _full_spec((2 * H,)),
                _full_spec((2 * H, H)), _full_spec((H,))],
      out_specs=_row_spec(H),
      out_shape=f32,
  )
  return out_mlp(tmps[0], tmps[1], tmps[2], tmps[3], wo1, bo1, Wo2, bo2)
